# Initial kernel scaffold; baseline (speedup 1.0000x reference)
import jax
import jax.numpy as jnp
from jax.experimental import pallas as pl


def kernel(x, gW1, gb1, gW2, gb2, gW3, gb3, iW1, ib1, iW2, ib2, iW3, ib3, dW1, db1, dW2, db2, dW3, db3):
    raise NotImplementedError("write your pallas kernel here")


# speedup vs baseline: 1.7793x; 1.7793x over previous
import functools

import jax
import jax.numpy as jnp
from jax import lax
from jax.experimental import pallas as pl

X_DIM = 3
R_DIM = 16
U_DIM = 64
V_DIM = 512
H_DIM = 64
B = 8
N = 8192

NB1 = 1024
NB2 = 1024


def _pos_bits(j, nb):
    nidx = lax.broadcasted_iota(jnp.int32, (nb, R_DIM), 0) + j * nb
    kexp = lax.broadcasted_iota(jnp.int32, (nb, R_DIM), 1)
    return (lax.shift_right_logical(nidx, kexp) & 1).astype(jnp.float32)


def _pass1_body(x_ref, gW1_ref, gb1_ref, gW2_ref, gb2_ref, hsum_ref):
    j = pl.program_id(1)
    xb = x_ref[0]
    r = _pos_bits(j, NB1)
    h = jax.nn.relu(jnp.concatenate([xb, r], axis=-1) @ gW1_ref[...] + gb1_ref[...])
    h = jax.nn.relu(h @ gW2_ref[...] + gb2_ref[...])
    part = jnp.sum(h, axis=0)[None, None, :]

    @pl.when(j == 0)
    def _():
        hsum_ref[...] = jnp.zeros_like(hsum_ref)

    hsum_ref[...] += part


def _pass2_body(x_ref, hsum_ref, gW3_ref, gb3_ref,
                iW1_ref, ib1_ref, iW2_ref, ib2_ref, iW3_ref, ib3_ref,
                dW1_ref, db1_ref, dW2_ref, db2_ref, dW3_ref, db3_ref,
                xloc_ref, vhard_ref):
    j = pl.program_id(1)
    xb = x_ref[0]
    r = _pos_bits(j, NB2)

    u = (hsum_ref[0] * (1.0 / N)) @ gW3_ref[...] + gb3_ref[...]
    ue = jnp.broadcast_to(u, (NB2, U_DIM))

    h = jax.nn.relu(jnp.concatenate([xb, r, ue], axis=-1) @ iW1_ref[...] + ib1_ref[...])
    h = jax.nn.relu(h @ iW2_ref[...] + ib2_ref[...])
    logits = h @ iW3_ref[...] + ib3_ref[...]

    mx = jnp.max(logits, axis=-1, keepdims=True)
    cidx = lax.broadcasted_iota(jnp.int32, (NB2, V_DIM), 1)
    idx = jnp.min(jnp.where(logits == mx, cidx, V_DIM), axis=-1, keepdims=True)
    onehot = (cidx == idx).astype(jnp.float32)
    vhard_ref[0] = onehot

    h = jax.nn.relu(jnp.concatenate([r, ue, onehot], axis=-1) @ dW1_ref[...] + db1_ref[...])
    h = jax.nn.relu(h @ dW2_ref[...] + db2_ref[...])
    xloc_ref[0] = h @ dW3_ref[...] + db3_ref[...]


def _full(spec):
    return pl.BlockSpec(spec, lambda b, j: tuple(0 for _ in spec))


@jax.jit
def kernel(x, gW1, gb1, gW2, gb2, gW3, gb3,
           iW1, ib1, iW2, ib2, iW3, ib3,
           dW1, db1, dW2, db2, dW3, db3):
    xspec1 = pl.BlockSpec((1, NB1, X_DIM), lambda b, j: (b, j, 0))
    hsum = pl.pallas_call(
        _pass1_body,
        grid=(B, N // NB1),
        in_specs=[xspec1, _full(gW1.shape), _full(gb1.shape),
                  _full(gW2.shape), _full(gb2.shape)],
        out_specs=pl.BlockSpec((1, 1, H_DIM), lambda b, j: (b, 0, 0)),
        out_shape=jax.ShapeDtypeStruct((B, 1, H_DIM), jnp.float32),
    )(x, gW1, gb1, gW2, gb2)

    xspec2 = pl.BlockSpec((1, NB2, X_DIM), lambda b, j: (b, j, 0))
    hsumspec = pl.BlockSpec((1, 1, H_DIM), lambda b, j: (b, 0, 0))
    x_loc, v_hard = pl.pallas_call(
        _pass2_body,
        grid=(B, N // NB2),
        in_specs=[xspec2, hsumspec, _full(gW3.shape), _full(gb3.shape),
                  _full(iW1.shape), _full(ib1.shape), _full(iW2.shape),
                  _full(ib2.shape), _full(iW3.shape), _full(ib3.shape),
                  _full(dW1.shape), _full(db1.shape), _full(dW2.shape),
                  _full(db2.shape), _full(dW3.shape), _full(db3.shape)],
        out_specs=[pl.BlockSpec((1, NB2, X_DIM), lambda b, j: (b, j, 0)),
                   pl.BlockSpec((1, NB2, V_DIM), lambda b, j: (b, j, 0))],
        out_shape=[jax.ShapeDtypeStruct((B, N, X_DIM), jnp.float32),
                   jax.ShapeDtypeStruct((B, N, V_DIM), jnp.float32)],
    )(x, hsum, gW3, gb3, iW1, ib1, iW2, ib2, iW3, ib3,
      dW1, db1, dW2, db2, dW3, db3)
    return (x_loc, v_hard)


# speedup vs baseline: 3.2845x; 1.8459x over previous
import jax
import jax.numpy as jnp
from jax import lax
from jax.experimental import pallas as pl

X_DIM = 3
R_DIM = 16
U_DIM = 64
V_DIM = 512
H_DIM = 64
B = 8
N = 8192

NB1 = 2048
NB2 = 2048


def _pass1_body(x_ref, r_ref, gW1_ref, gb1_ref, gW2_ref, gb2_ref,
                gW3_ref, gb3_ref, dW1u_ref, db1_ref,
                u_ref, urow_ref, hsum_ref):
    j = pl.program_id(1)
    xb = x_ref[0]
    h = jax.nn.relu(jnp.concatenate([xb, r_ref[...]], axis=-1) @ gW1_ref[...]
                    + gb1_ref[...])
    h = jax.nn.relu(h @ gW2_ref[...] + gb2_ref[...])
    part = jnp.sum(h, axis=0)[None, None, :]

    @pl.when(j == 0)
    def _():
        hsum_ref[...] = jnp.zeros_like(hsum_ref)

    hsum_ref[...] += part

    @pl.when(j == pl.num_programs(1) - 1)
    def _():
        u = (hsum_ref[0] * (1.0 / N)) @ gW3_ref[...] + gb3_ref[...]
        u_ref[0] = u
        urow_ref[0] = u @ dW1u_ref[...] + db1_ref[...]


def _pass2_body(x_ref, r_ref, u_ref, urow_ref,
                iW1_ref, ib1_ref, iW2_ref, ib2_ref, iW3_ref, ib3_ref,
                dW1r_ref, dW1v_ref, dW2_ref, db2_ref, dW3_ref, db3_ref,
                xloc_ref, vhard_ref):
    xb = x_ref[0]
    r = r_ref[...]
    ue = jnp.broadcast_to(u_ref[0], (NB2, U_DIM))

    h = jax.nn.relu(jnp.concatenate([xb, r, ue], axis=-1) @ iW1_ref[...] + ib1_ref[...])
    h = jax.nn.relu(h @ iW2_ref[...] + ib2_ref[...])
    logits = h @ iW3_ref[...] + ib3_ref[...]

    mx = jnp.max(logits, axis=-1, keepdims=True)
    onehot = (logits == mx).astype(jnp.float32)
    vhard_ref[0] = onehot

    h = jax.nn.relu(r @ dW1r_ref[...] + onehot @ dW1v_ref[...] + urow_ref[0])
    h = jax.nn.relu(h @ dW2_ref[...] + db2_ref[...])
    xloc_ref[0] = h @ dW3_ref[...] + db3_ref[...]


def _full(shape):
    return pl.BlockSpec(shape, lambda b, j: tuple(0 for _ in shape))


@jax.jit
def kernel(x, gW1, gb1, gW2, gb2, gW3, gb3,
           iW1, ib1, iW2, ib2, iW3, ib3,
           dW1, db1, dW2, db2, dW3, db3):
    pos = jnp.arange(N, dtype=jnp.int32)
    r_all = ((pos[:, None] >> jnp.arange(R_DIM, dtype=jnp.int32)[None, :]) & 1
             ).astype(jnp.float32)
    dW1r = dW1[:R_DIM]
    dW1u = dW1[R_DIM:R_DIM + U_DIM]
    dW1v = dW1[R_DIM + U_DIM:]

    bspec = pl.BlockSpec((1, 1, H_DIM), lambda b, j: (b, 0, 0))
    u, urow, _ = pl.pallas_call(
        _pass1_body,
        grid=(B, N // NB1),
        in_specs=[pl.BlockSpec((1, NB1, X_DIM), lambda b, j: (b, j, 0)),
                  pl.BlockSpec((NB1, R_DIM), lambda b, j: (j, 0)),
                  _full(gW1.shape), _full(gb1.shape),
                  _full(gW2.shape), _full(gb2.shape),
                  _full(gW3.shape), _full(gb3.shape),
                  _full(dW1u.shape), _full(db1.shape)],
        out_specs=[bspec, bspec, bspec],
        out_shape=[jax.ShapeDtypeStruct((B, 1, H_DIM), jnp.float32),
                   jax.ShapeDtypeStruct((B, 1, H_DIM), jnp.float32),
                   jax.ShapeDtypeStruct((B, 1, H_DIM), jnp.float32)],
    )(x, r_all, gW1, gb1, gW2, gb2, gW3, gb3, dW1u, db1)

    x_loc, v_hard = pl.pallas_call(
        _pass2_body,
        grid=(B, N // NB2),
        in_specs=[pl.BlockSpec((1, NB2, X_DIM), lambda b, j: (b, j, 0)),
                  pl.BlockSpec((NB2, R_DIM), lambda b, j: (j, 0)),
                  bspec, bspec,
                  _full(iW1.shape), _full(ib1.shape), _full(iW2.shape),
                  _full(ib2.shape), _full(iW3.shape), _full(ib3.shape),
                  _full(dW1r.shape), _full(dW1v.shape), _full(dW2.shape),
                  _full(db2.shape), _full(dW3.shape), _full(db3.shape)],
        out_specs=[pl.BlockSpec((1, NB2, X_DIM), lambda b, j: (b, j, 0)),
                   pl.BlockSpec((1, NB2, V_DIM), lambda b, j: (b, j, 0))],
        out_shape=[jax.ShapeDtypeStruct((B, N, X_DIM), jnp.float32),
                   jax.ShapeDtypeStruct((B, N, V_DIM), jnp.float32)],
    )(x, r_all, u, urow, iW1, ib1, iW2, ib2, iW3, ib3,
      dW1r, dW1v, dW2, db2, dW3, db3)
    return (x_loc, v_hard)


# speedup vs baseline: 3.2861x; 1.0005x over previous
import jax
import jax.numpy as jnp
from jax import lax
from jax.experimental import pallas as pl

X_DIM = 3
R_DIM = 16
U_DIM = 64
V_DIM = 512
H_DIM = 64
B = 8
N = 8192

NB1 = 2048
NB2 = 512


def _pass1_body(x_ref, r_ref, gW1_ref, gb1_ref, gW2_ref, gb2_ref,
                gW3_ref, gb3_ref, dW1u_ref, db1_ref,
                u_ref, urow_ref, hsum_ref):
    j = pl.program_id(0)
    xb = x_ref[...].reshape(B * NB1, X_DIM)
    rb = jnp.broadcast_to(r_ref[...][None], (B, NB1, R_DIM)).reshape(B * NB1, R_DIM)
    h = jax.nn.relu(jnp.concatenate([xb, rb], axis=-1) @ gW1_ref[...]
                    + gb1_ref[...])
    h = jax.nn.relu(h @ gW2_ref[...] + gb2_ref[...])
    part = jnp.sum(h.reshape(B, NB1, H_DIM), axis=1)

    @pl.when(j == 0)
    def _():
        hsum_ref[...] = jnp.zeros_like(hsum_ref)

    hsum_ref[...] += part

    @pl.when(j == pl.num_programs(0) - 1)
    def _():
        u = (hsum_ref[...] * (1.0 / N)) @ gW3_ref[...] + gb3_ref[...]
        u_ref[...] = u
        urow_ref[...] = u @ dW1u_ref[...] + db1_ref[...]


def _pass2_body(x_ref, r_ref, u_ref, urow_ref,
                iW1_ref, ib1_ref, iW2_ref, ib2_ref, iW3_ref, ib3_ref,
                dW1r_ref, dW1v_ref, dW2_ref, db2_ref, dW3_ref, db3_ref,
                xloc_ref, vhard_ref):
    M = B * NB2
    xb = x_ref[...].reshape(M, X_DIM)
    rb = jnp.broadcast_to(r_ref[...][None], (B, NB2, R_DIM)).reshape(M, R_DIM)
    ue = jnp.broadcast_to(u_ref[...][:, None, :], (B, NB2, U_DIM)).reshape(M, U_DIM)

    h = jax.nn.relu(jnp.concatenate([xb, rb, ue], axis=-1) @ iW1_ref[...] + ib1_ref[...])
    h = jax.nn.relu(h @ iW2_ref[...] + ib2_ref[...])
    logits = h @ iW3_ref[...] + ib3_ref[...]

    mx = jnp.max(logits, axis=-1, keepdims=True)
    onehot = (logits == mx).astype(jnp.float32)
    vhard_ref[...] = onehot.reshape(B, NB2, V_DIM)

    ur = jnp.broadcast_to(urow_ref[...][:, None, :], (B, NB2, H_DIM)).reshape(M, H_DIM)
    h = jax.nn.relu(rb @ dW1r_ref[...] + onehot @ dW1v_ref[...] + ur)
    h = jax.nn.relu(h @ dW2_ref[...] + db2_ref[...])
    xloc_ref[...] = (h @ dW3_ref[...] + db3_ref[...]).reshape(B, NB2, X_DIM)


def _full(shape):
    return pl.BlockSpec(shape, lambda j: tuple(0 for _ in shape))


@jax.jit
def kernel(x, gW1, gb1, gW2, gb2, gW3, gb3,
           iW1, ib1, iW2, ib2, iW3, ib3,
           dW1, db1, dW2, db2, dW3, db3):
    pos = jnp.arange(N, dtype=jnp.int32)
    r_all = ((pos[:, None] >> jnp.arange(R_DIM, dtype=jnp.int32)[None, :]) & 1
             ).astype(jnp.float32)
    dW1r = dW1[:R_DIM]
    dW1u = dW1[R_DIM:R_DIM + U_DIM]
    dW1v = dW1[R_DIM + U_DIM:]

    u, urow, _ = pl.pallas_call(
        _pass1_body,
        grid=(N // NB1,),
        in_specs=[pl.BlockSpec((B, NB1, X_DIM), lambda j: (0, j, 0)),
                  pl.BlockSpec((NB1, R_DIM), lambda j: (j, 0)),
                  _full(gW1.shape), _full(gb1.shape),
                  _full(gW2.shape), _full(gb2.shape),
                  _full(gW3.shape), _full(gb3.shape),
                  _full(dW1u.shape), _full(db1.shape)],
        out_specs=[_full((B, H_DIM)), _full((B, H_DIM)), _full((B, H_DIM))],
        out_shape=[jax.ShapeDtypeStruct((B, H_DIM), jnp.float32),
                   jax.ShapeDtypeStruct((B, H_DIM), jnp.float32),
                   jax.ShapeDtypeStruct((B, H_DIM), jnp.float32)],
    )(x, r_all, gW1, gb1, gW2, gb2, gW3, gb3, dW1u, db1)

    x_loc, v_hard = pl.pallas_call(
        _pass2_body,
        grid=(N // NB2,),
        in_specs=[pl.BlockSpec((B, NB2, X_DIM), lambda j: (0, j, 0)),
                  pl.BlockSpec((NB2, R_DIM), lambda j: (j, 0)),
                  _full((B, H_DIM)), _full((B, H_DIM)),
                  _full(iW1.shape), _full(ib1.shape), _full(iW2.shape),
                  _full(ib2.shape), _full(iW3.shape), _full(ib3.shape),
                  _full(dW1r.shape), _full(dW1v.shape), _full(dW2.shape),
                  _full(db2.shape), _full(dW3.shape), _full(db3.shape)],
        out_specs=[pl.BlockSpec((B, NB2, X_DIM), lambda j: (0, j, 0)),
                   pl.BlockSpec((B, NB2, V_DIM), lambda j: (0, j, 0))],
        out_shape=[jax.ShapeDtypeStruct((B, N, X_DIM), jnp.float32),
                   jax.ShapeDtypeStruct((B, N, V_DIM), jnp.float32)],
    )(x, r_all, u, urow, iW1, ib1, iW2, ib2, iW3, ib3,
      dW1r, dW1v, dW2, db2, dW3, db3)
    return (x_loc, v_hard)
